# baseline (speedup 1.0000x reference)
import functools

import jax
import jax.numpy as jnp
from jax import lax
from jax.experimental import pallas as pl
from jax.experimental.pallas import tpu as pltpu
from jax.experimental.pallas import tpu_sc as plsc

VOCAB = 1000000
PCA = 16
HIDDEN = 128
B = 16384

_info = plsc.get_sparse_core_info()
_NC, _NS = _info.num_cores, _info.num_subcores
NW = _NC * _NS
BPW = B // NW
_GROUPS = BPW // 16


def _extract(blk_v, lo_v, out_v, g):
    lanes = lax.iota(jnp.int32, 16)
    row = g * 16 + lanes
    lo16 = lo_v[pl.ds(g * 16, 16)]
    col0 = lo16 * PCA
    for k in range(PCA):
        vals = plsc.load_gather(blk_v, [row, col0 + k])
        plsc.store_scatter(out_v, [row * PCA + k], vals)


def _gather_body(xh_hbm, xl_hbm, c_hbm, h_hbm, outc_hbm, outh_hbm,
                 idxh_v, idxl_v, blk_v, out_v, sem):
    wid = lax.axis_index("s") * _NC + lax.axis_index("c")
    base = wid * BPW
    pltpu.sync_copy(xh_hbm.at[pl.ds(base, BPW)], idxh_v)
    pltpu.sync_copy(xl_hbm.at[pl.ds(base, BPW)], idxl_v)

    for tbl_hbm, out_hbm in ((c_hbm, outc_hbm), (h_hbm, outh_hbm)):
        pltpu.async_copy(tbl_hbm.at[idxh_v], blk_v, sem).wait()

        def body(g, carry):
            _extract(blk_v, idxl_v, out_v, g)
            return carry

        lax.fori_loop(0, _GROUPS, body, 0)
        pltpu.sync_copy(out_v, out_hbm.at[pl.ds(base * PCA, BPW * PCA)])


_sc_gather = functools.partial(
    pl.kernel,
    mesh=plsc.VectorSubcoreMesh(core_axis_name="c", subcore_axis_name="s"),
    out_type=[jax.ShapeDtypeStruct((B * PCA,), jnp.float32),
              jax.ShapeDtypeStruct((B * PCA,), jnp.float32)],
    scratch_types=[
        pltpu.VMEM((BPW,), jnp.int32),
        pltpu.VMEM((BPW,), jnp.int32),
        pltpu.VMEM((BPW, 8 * PCA), jnp.float32),
        pltpu.VMEM((BPW * PCA,), jnp.float32),
        pltpu.SemaphoreType.DMA,
    ],
    compiler_params=pltpu.CompilerParams(needs_layout_passes=False),
)(_gather_body)


_BB = 2048


def _proj_body(ec_ref, eh_ref, wc_ref, wh_ref, bc_ref, bh_ref,
               oc_ref, oh_ref):
    oc_ref[...] = (
        jnp.dot(ec_ref[...], wc_ref[...], preferred_element_type=jnp.float32)
        + bc_ref[...])
    oh_ref[...] = (
        jnp.dot(eh_ref[...], wh_ref[...], preferred_element_type=jnp.float32)
        + bh_ref[...])


def _project(emb_c, emb_h, Wct, Wht, bc2, bh2):
    grid = B // _BB
    return pl.pallas_call(
        _proj_body,
        grid=(grid,),
        in_specs=[
            pl.BlockSpec((_BB, PCA), lambda i: (i, 0)),
            pl.BlockSpec((_BB, PCA), lambda i: (i, 0)),
            pl.BlockSpec((PCA, HIDDEN), lambda i: (0, 0)),
            pl.BlockSpec((PCA, HIDDEN), lambda i: (0, 0)),
            pl.BlockSpec((1, HIDDEN), lambda i: (0, 0)),
            pl.BlockSpec((1, HIDDEN), lambda i: (0, 0)),
        ],
        out_specs=[
            pl.BlockSpec((_BB, HIDDEN), lambda i: (i, 0)),
            pl.BlockSpec((_BB, HIDDEN), lambda i: (i, 0)),
        ],
        out_shape=[
            jax.ShapeDtypeStruct((B, HIDDEN), jnp.float32),
            jax.ShapeDtypeStruct((B, HIDDEN), jnp.float32),
        ],
    )(emb_c, emb_h, Wct, Wht, bc2, bh2)


def kernel(x, c_table, h_table, Wc, bc, Wh, bh):
    xi = x.astype(jnp.int32)
    xh = xi >> 3
    xl = xi & 7
    ct = c_table.reshape(VOCAB // 8, 8 * PCA)
    ht = h_table.reshape(VOCAB // 8, 8 * PCA)
    ec_flat, eh_flat = _sc_gather(xh, xl, ct, ht)
    oc, oh = _project(ec_flat.reshape(B, PCA), eh_flat.reshape(B, PCA),
                      Wc.T, Wh.T,
                      bc.reshape(1, HIDDEN), bh.reshape(1, HIDDEN))
    return (oc.reshape(1, B, HIDDEN), oh.reshape(1, B, HIDDEN))
